# baseline (speedup 1.0000x reference)
import functools

import jax
import jax.numpy as jnp
from jax import lax
from jax.experimental import pallas as pl
from jax.experimental.pallas import tpu as pltpu
from jax.experimental.pallas import tpu_sc as plsc

_L = 199
_D = 128
_NM = 32
_B = 1024
_NC = 2
_NS = 16
_NW = _NC * _NS
_BPW = _B // _NW
_NCH = _D // 16


def _sc_body(xt_hbm, xp_hbm, masks_hbm, ind_hbm, out_hbm,
             xt_v, xp_v, out_v, masks_v, ind_v):
    wid = lax.axis_index("s") * _NC + lax.axis_index("c")
    pltpu.sync_copy(masks_hbm, masks_v)
    pltpu.sync_copy(ind_hbm, ind_v)
    lanes = lax.iota(jnp.int32, 16)

    def one_token(t, carry):
        maxv = xt_v[t, pl.ds(0, 16)]
        cidx = jnp.zeros((16,), jnp.int32)
        for c in range(1, _NCH):
            v = xt_v[t, pl.ds(c * 16, 16)]
            better = v > maxv
            maxv = jnp.where(better, v, maxv)
            cidx = jnp.where(better, c, cidx)
        gmax = jax.lax.reduce_max(maxv, (0,))
        dcand = jnp.where(maxv == gmax, cidx * 16 + lanes, _D)
        bestd = jax.lax.reduce_min(dcand, (0,))
        ix2 = ind_v[bestd, pl.ds(0, 16)][0]
        es = []
        s = jnp.zeros((16,), jnp.float32)
        for c in range(_NCH):
            m = masks_v[ix2, pl.ds(c * 16, 16)]
            p = xp_v[t, pl.ds(c * 16, 16)]
            e = jnp.exp(p) * m
            es.append(e)
            s = s + e
        total = jax.lax.reduce_sum(s, (0,))
        rinv = jnp.ones((16,), jnp.float32) / jnp.full((16,), total, jnp.float32)
        for c in range(_NCH):
            out_v[t, pl.ds(c * 16, 16)] = es[c] * rinv
        return carry

    def one_batch(b, carry):
        pltpu.sync_copy(xt_hbm.at[b], xt_v)
        pltpu.sync_copy(xp_hbm.at[b], xp_v)
        lax.fori_loop(0, _L, one_token, 0)
        pltpu.sync_copy(out_v, out_hbm.at[b])
        return carry

    lax.fori_loop(wid * _BPW, (wid + 1) * _BPW, one_batch, 0)


def kernel(x_true, x_pred, masks, ind_of_ind):
    mesh = plsc.VectorSubcoreMesh(core_axis_name="c", subcore_axis_name="s")
    f = functools.partial(
        pl.kernel,
        mesh=mesh,
        compiler_params=pltpu.CompilerParams(needs_layout_passes=False),
        out_type=jax.ShapeDtypeStruct((_B, _L, _D), jnp.float32),
        scratch_types=[
            pltpu.VMEM((_L, _D), jnp.float32),
            pltpu.VMEM((_L, _D), jnp.float32),
            pltpu.VMEM((_L, _D), jnp.float32),
            pltpu.VMEM((_NM, _D), jnp.float32),
            pltpu.VMEM((_D, 16), jnp.int32),
        ],
    )(_sc_body)
    ind_rep = jnp.broadcast_to(
        ind_of_ind.astype(jnp.int32).reshape(_D, 1), (_D, 16))
    return f(x_true, x_pred, masks, ind_rep)
